# baseline (speedup 1.0000x reference)
import functools

import jax
import jax.numpy as jnp
from jax import lax
from jax.experimental import pallas as pl
from jax.experimental.pallas import tpu as pltpu
from jax.experimental.pallas import tpu_sc as plsc

B = 16384
D = 32
L = 16
NW = 32
BPW = B // NW
CH = 128
NCH = BPW // CH
GPW = BPW // L
GPC = CH // L

_mesh = plsc.VectorSubcoreMesh(core_axis_name="c", subcore_axis_name="s")


@functools.partial(
    pl.kernel,
    out_type=jax.ShapeDtypeStruct((B,), jnp.float32),
    mesh=_mesh,
    compiler_params=pltpu.CompilerParams(
        needs_layout_passes=False, use_tc_tiling_on_sc=False),
    scratch_types=[
        pltpu.VMEM((NCH, CH), jnp.int32),
        pltpu.VMEM((NCH, CH), jnp.int32),
        pltpu.VMEM((BPW, D), jnp.float32),
        pltpu.VMEM((BPW, D), jnp.float32),
        pltpu.VMEM((BPW,), jnp.float32),
        pltpu.VMEM((L,), jnp.float32),
        pltpu.VMEM((L,), jnp.float32),
        pltpu.SemaphoreType.DMA,
    ],
)
def _sc_embed_dot(xu_hbm, xm_hbm, ut_hbm, mt_hbm, w_hbm, b_hbm, out_hbm,
                  idx_u, idx_m, urows, mrows, outv, wv, bv, sem):
    wid = lax.axis_index("s") * 2 + lax.axis_index("c")
    base = wid * BPW

    pltpu.sync_copy(xu_hbm.at[wid], idx_u)
    pltpu.sync_copy(xm_hbm.at[wid], idx_m)
    pltpu.sync_copy(w_hbm, wv)
    pltpu.sync_copy(b_hbm, bv)

    copies = []
    for j in range(NCH):
        copies.append(pltpu.async_copy(
            ut_hbm.at[idx_u.at[j]], urows.at[pl.ds(j * CH, CH)], sem))
        copies.append(pltpu.async_copy(
            mt_hbm.at[idx_m.at[j]], mrows.at[pl.ds(j * CH, CH)], sem))
    for c in copies:
        c.wait()

    wvec = wv[...]
    bvec = bv[...]
    iota = lax.broadcasted_iota(jnp.int32, (L,), 0)

    def group_body(g, carry):
        rv = g * L + iota
        acc = jnp.zeros((L,), dtype=jnp.float32)
        for d in range(D):
            dv = jnp.full((L,), d, dtype=jnp.int32)
            gu = plsc.load_gather(urows, [rv, dv])
            gm = plsc.load_gather(mrows, [rv, dv])
            acc = acc + gu * gm
        z = acc * wvec + bvec
        outv[pl.ds(g * L, L)] = 1.0 / (1.0 + jnp.exp(-z))
        return carry

    lax.fori_loop(0, GPW, group_body, 0)

    pltpu.sync_copy(outv, out_hbm.at[pl.ds(base, BPW)])


def kernel(x, user_table, movie_table, W, b):
    xi = x.astype(jnp.int32)
    xu = xi[0].reshape(NW, NCH, CH)
    xm = xi[1].reshape(NW, NCH, CH)
    w16 = jnp.broadcast_to(W.reshape(1), (L,)).astype(jnp.float32)
    b16 = jnp.broadcast_to(b.reshape(1), (L,)).astype(jnp.float32)
    out = _sc_embed_dot(xu, xm, user_table, movie_table, w16, b16)
    return out.reshape(B, 1)
